# Initial kernel scaffold; baseline (speedup 1.0000x reference)
import jax
import jax.numpy as jnp
from jax.experimental import pallas as pl


def kernel(x, keywords_type, type_embedding):
    raise NotImplementedError("write your pallas kernel here")


# speedup vs baseline: 2.0552x; 2.0552x over previous
import jax
import jax.numpy as jnp
from jax.experimental import pallas as pl
from jax.experimental.pallas import tpu as pltpu

D_MODEL = 1024
N_ROWS = 8
BLOCK_T = 2048


def _body(idx_ref, table_ref, x_ref, out_ref):
    idx = idx_ref[...]
    rows = jax.lax.broadcasted_iota(jnp.int32, (BLOCK_T, N_ROWS), 1)
    onehot = (idx == rows).astype(jnp.float32)
    emb = jnp.dot(
        onehot,
        table_ref[...],
        preferred_element_type=jnp.float32,
        precision=jax.lax.Precision.HIGHEST,
    )
    out_ref[...] = x_ref[...] + emb


def kernel(x, keywords_type, type_embedding):
    b, s, d = x.shape
    n = b * s
    x2 = x.reshape(n, d)
    idx2 = keywords_type.astype(jnp.int32).reshape(n, 1)
    table = jnp.zeros((N_ROWS, d), jnp.float32).at[: type_embedding.shape[0]].set(
        type_embedding
    )

    grid = (n // BLOCK_T,)
    out = pl.pallas_call(
        _body,
        grid=grid,
        in_specs=[
            pl.BlockSpec((BLOCK_T, 1), lambda i: (i, 0)),
            pl.BlockSpec((N_ROWS, d), lambda i: (0, 0)),
            pl.BlockSpec((BLOCK_T, d), lambda i: (i, 0)),
        ],
        out_specs=pl.BlockSpec((BLOCK_T, d), lambda i: (i, 0)),
        out_shape=jax.ShapeDtypeStruct((n, d), jnp.float32),
    )(idx2, table, x2)
    return out.reshape(b, s, d)
